# Initial kernel scaffold; baseline (speedup 1.0000x reference)
import jax
import jax.numpy as jnp
from jax.experimental import pallas as pl


def kernel(input, token_values):
    raise NotImplementedError("write your pallas kernel here")


# speedup vs baseline: 445.7242x; 445.7242x over previous
import functools
import jax
import jax.numpy as jnp
from jax import lax
from jax.experimental import pallas as pl
from jax.experimental.pallas import tpu as pltpu
from jax.experimental.pallas import tpu_sc as plsc

LANES = 16
UNROLL = 4
CHUNK = 8192


def _search_block(xbuf, obuf, edges_v, lane0, n_tokens):
    x = xbuf[pl.ds(lane0, LANES)]
    pos = jnp.zeros((LANES,), jnp.int32)
    step = n_tokens // 2
    while step >= 2:
        probe = pos + (step - 1)
        e = plsc.load_gather(edges_v, [probe])
        pos = jnp.where(e < x, pos + step, pos)
        step //= 2
    eprobe = plsc.load_gather(edges_v, [pos])
    m = eprobe < x
    pos = pos + m.astype(jnp.int32)
    other_idx = jnp.where(m, pos, (pos + (n_tokens - 1)) & (n_tokens - 1))
    eother = plsc.load_gather(edges_v, [other_idx])
    e0 = jnp.where(m, eother, eprobe)
    em1 = jnp.where(m, eprobe, eother)
    d0 = jnp.abs(e0 - x)
    d1 = jnp.abs(x - em1)
    tok = pos - (d1 < d0).astype(jnp.int32)
    obuf[pl.ds(lane0, LANES)] = tok


def _sc_body(n_tokens, per_w, n_chunks, inp_hbm, tok_hbm, out_hbm,
             edges_v, xbuf, obuf):
    wid = lax.axis_index("s") * 2 + lax.axis_index("c")
    base = wid * per_w
    pltpu.sync_copy(tok_hbm, edges_v)

    def chunk_body(ci, carry):
        off = base + ci * CHUNK
        pltpu.sync_copy(inp_hbm.at[pl.ds(off, CHUNK)], xbuf)

        def vbody(vi, carry2):
            for u in range(UNROLL):
                _search_block(xbuf, obuf, edges_v,
                              (vi * UNROLL + u) * LANES, n_tokens)
            return carry2

        lax.fori_loop(0, CHUNK // (LANES * UNROLL), vbody, 0)
        pltpu.sync_copy(obuf, out_hbm.at[pl.ds(off, CHUNK)])
        return carry

    lax.fori_loop(0, n_chunks, chunk_body, 0)


@jax.jit
def kernel(input, token_values):
    n_values = input.shape[0]
    n_tokens = token_values.shape[0]
    n_workers = 32
    per_w = n_values // n_workers
    n_chunks = per_w // CHUNK

    mesh = plsc.VectorSubcoreMesh(core_axis_name="c", subcore_axis_name="s")
    k = functools.partial(
        pl.kernel,
        out_type=jax.ShapeDtypeStruct((n_values,), jnp.int32),
        mesh=mesh,
        scratch_types=[
            pltpu.VMEM((n_tokens,), jnp.float32),
            pltpu.VMEM((CHUNK,), jnp.float32),
            pltpu.VMEM((CHUNK,), jnp.int32),
        ],
        compiler_params=pltpu.CompilerParams(needs_layout_passes=False),
    )(functools.partial(_sc_body, n_tokens, per_w, n_chunks))
    return k(input, token_values)


# speedup vs baseline: 959.3557x; 2.1524x over previous
import functools
import jax
import jax.numpy as jnp
from jax import lax
from jax.experimental import pallas as pl
from jax.experimental.pallas import tpu as pltpu
from jax.experimental.pallas import tpu_sc as plsc

LANES = 16
UNROLL = 8
CHUNK = 8192


def _search_block(xbuf, obuf, edges_v, lane0, n_tokens):
    x = xbuf[pl.ds(lane0, LANES)]
    pos = jnp.zeros((LANES,), jnp.int32)
    step = n_tokens // 2
    while step >= 2:
        probe = pos + (step - 1)
        e = plsc.load_gather(edges_v, [probe])
        pos = jnp.where(e < x, pos + step, pos)
        step //= 2
    eprobe = plsc.load_gather(edges_v, [pos])
    m = eprobe < x
    pos = pos + m.astype(jnp.int32)
    other_idx = jnp.where(m, pos, (pos + (n_tokens - 1)) & (n_tokens - 1))
    eother = plsc.load_gather(edges_v, [other_idx])
    e0 = jnp.where(m, eother, eprobe)
    em1 = jnp.where(m, eprobe, eother)
    d0 = jnp.abs(e0 - x)
    d1 = jnp.abs(x - em1)
    tok = pos - (d1 < d0).astype(jnp.int32)
    obuf[pl.ds(lane0, LANES)] = tok


def _sc_body(n_tokens, per_w, n_chunks, inp_hbm, tok_hbm, out_hbm,
             edges_v, xbuf, obuf):
    wid = lax.axis_index("s") * 2 + lax.axis_index("c")
    base = wid * per_w
    pltpu.sync_copy(tok_hbm, edges_v)

    def chunk_body(ci, carry):
        off = base + ci * CHUNK
        pltpu.sync_copy(inp_hbm.at[pl.ds(off, CHUNK)], xbuf)

        @plsc.parallel_loop(0, CHUNK, step=LANES, unroll=UNROLL)
        def vbody(lane0):
            _search_block(xbuf, obuf, edges_v, lane0, n_tokens)

        pltpu.sync_copy(obuf, out_hbm.at[pl.ds(off, CHUNK)])
        return carry

    lax.fori_loop(0, n_chunks, chunk_body, 0)


@jax.jit
def kernel(input, token_values):
    n_values = input.shape[0]
    n_tokens = token_values.shape[0]
    n_workers = 32
    per_w = n_values // n_workers
    n_chunks = per_w // CHUNK

    mesh = plsc.VectorSubcoreMesh(core_axis_name="c", subcore_axis_name="s")
    k = functools.partial(
        pl.kernel,
        out_type=jax.ShapeDtypeStruct((n_values,), jnp.int32),
        mesh=mesh,
        scratch_types=[
            pltpu.VMEM((n_tokens,), jnp.float32),
            pltpu.VMEM((CHUNK,), jnp.float32),
            pltpu.VMEM((CHUNK,), jnp.int32),
        ],
        compiler_params=pltpu.CompilerParams(needs_layout_passes=False),
    )(functools.partial(_sc_body, n_tokens, per_w, n_chunks))
    return k(input, token_values)


# speedup vs baseline: 1029.7729x; 1.0734x over previous
import functools
import jax
import jax.numpy as jnp
from jax import lax
from jax.experimental import pallas as pl
from jax.experimental.pallas import tpu as pltpu
from jax.experimental.pallas import tpu_sc as plsc

LANES = 16
UNROLL = 2
GROUP = 8
CHUNK = 8192


def _search(x, edges_v, n_tokens):
    pos = jnp.zeros((LANES,), jnp.int32)
    step = n_tokens // 2
    while step >= 2:
        probe = pos + (step - 1)
        e = plsc.load_gather(edges_v, [probe])
        pos = jnp.where(e < x, pos + step, pos)
        step //= 2
    eprobe = plsc.load_gather(edges_v, [pos])
    m = eprobe < x
    pos = pos + m.astype(jnp.int32)
    other_idx = jnp.where(m, pos, (pos + (n_tokens - 1)) & (n_tokens - 1))
    eother = plsc.load_gather(edges_v, [other_idx])
    e0 = jnp.where(m, eother, eprobe)
    em1 = jnp.where(m, eprobe, eother)
    d0 = jnp.abs(e0 - x)
    d1 = jnp.abs(x - em1)
    return pos - (d1 < d0).astype(jnp.int32)


def _sc_body(n_tokens, per_w, n_chunks, inp_hbm, tok_hbm, out_hbm,
             edges_v, xbuf, obuf):
    wid = lax.axis_index("s") * 2 + lax.axis_index("c")
    base = wid * per_w
    pltpu.sync_copy(tok_hbm, edges_v)

    def chunk_body(ci, carry):
        off = base + ci * CHUNK
        pltpu.sync_copy(inp_hbm.at[pl.ds(off, CHUNK)], xbuf)

        @plsc.parallel_loop(0, CHUNK, step=LANES * GROUP, unroll=UNROLL)
        def vbody(g0):
            xs = [xbuf[pl.ds(g0 + k * LANES, LANES)] for k in range(GROUP)]
            toks = [_search(xs[k], edges_v, n_tokens) for k in range(GROUP)]
            for k in range(GROUP):
                obuf[pl.ds(g0 + k * LANES, LANES)] = toks[k]

        pltpu.sync_copy(obuf, out_hbm.at[pl.ds(off, CHUNK)])
        return carry

    lax.fori_loop(0, n_chunks, chunk_body, 0)


@jax.jit
def kernel(input, token_values):
    n_values = input.shape[0]
    n_tokens = token_values.shape[0]
    n_workers = 32
    per_w = n_values // n_workers
    n_chunks = per_w // CHUNK

    mesh = plsc.VectorSubcoreMesh(core_axis_name="c", subcore_axis_name="s")
    k = functools.partial(
        pl.kernel,
        out_type=jax.ShapeDtypeStruct((n_values,), jnp.int32),
        mesh=mesh,
        scratch_types=[
            pltpu.VMEM((n_tokens,), jnp.float32),
            pltpu.VMEM((CHUNK,), jnp.float32),
            pltpu.VMEM((CHUNK,), jnp.int32),
        ],
        compiler_params=pltpu.CompilerParams(needs_layout_passes=False),
    )(functools.partial(_sc_body, n_tokens, per_w, n_chunks))
    return k(input, token_values)


# speedup vs baseline: 1061.9266x; 1.0312x over previous
import functools
import jax
import jax.numpy as jnp
from jax import lax
from jax.experimental import pallas as pl
from jax.experimental.pallas import tpu as pltpu
from jax.experimental.pallas import tpu_sc as plsc

LANES = 16
UNROLL = 2
GROUP = 8
CHUNK = 16384


def _search(x, edges_v, n_tokens):
    pos = jnp.zeros((LANES,), jnp.int32)
    step = n_tokens // 2
    while step >= 2:
        probe = pos + (step - 1)
        e = plsc.load_gather(edges_v, [probe])
        pos = jnp.where(e < x, pos + step, pos)
        step //= 2
    eprobe = plsc.load_gather(edges_v, [pos])
    m = eprobe < x
    pos = pos + m.astype(jnp.int32)
    other_idx = jnp.where(m, pos, (pos + (n_tokens - 1)) & (n_tokens - 1))
    eother = plsc.load_gather(edges_v, [other_idx])
    e0 = jnp.where(m, eother, eprobe)
    em1 = jnp.where(m, eprobe, eother)
    d0 = jnp.abs(e0 - x)
    d1 = jnp.abs(x - em1)
    return pos - (d1 < d0).astype(jnp.int32)


def _sc_body(n_tokens, per_w, n_chunks, inp_hbm, tok_hbm, out_hbm,
             edges_v, xb0, xb1, ob0, ob1, si0, si1, so0, so1):
    wid = lax.axis_index("s") * 2 + lax.axis_index("c")
    base = wid * per_w
    pltpu.sync_copy(tok_hbm, edges_v)

    def in_slice(ci):
        return inp_hbm.at[pl.ds(base + ci * CHUNK, CHUNK)]

    def out_slice(ci):
        return out_hbm.at[pl.ds(base + ci * CHUNK, CHUNK)]

    def compute(xbuf, obuf):
        @plsc.parallel_loop(0, CHUNK, step=LANES * GROUP, unroll=UNROLL)
        def vbody(g0):
            xs = [xbuf[pl.ds(g0 + k * LANES, LANES)] for k in range(GROUP)]
            toks = [_search(xs[k], edges_v, n_tokens) for k in range(GROUP)]
            for k in range(GROUP):
                obuf[pl.ds(g0 + k * LANES, LANES)] = toks[k]

    pltpu.async_copy(in_slice(0), xb0, si0)
    nsuper = n_chunks // 2

    def super_body(i, carry):
        ci0 = 2 * i
        pltpu.async_copy(in_slice(ci0 + 1), xb1, si1)
        pltpu.make_async_copy(in_slice(ci0), xb0, si0).wait()

        @pl.when(i > 0)
        def _():
            pltpu.make_async_copy(ob0, out_slice(ci0 - 2), so0).wait()

        compute(xb0, ob0)
        pltpu.async_copy(ob0, out_slice(ci0), so0)

        @pl.when(i < nsuper - 1)
        def _():
            pltpu.async_copy(in_slice(ci0 + 2), xb0, si0)

        pltpu.make_async_copy(in_slice(ci0 + 1), xb1, si1).wait()

        @pl.when(i > 0)
        def _():
            pltpu.make_async_copy(ob1, out_slice(ci0 - 1), so1).wait()

        compute(xb1, ob1)
        pltpu.async_copy(ob1, out_slice(ci0 + 1), so1)
        return carry

    lax.fori_loop(0, nsuper, super_body, 0)
    pltpu.make_async_copy(ob0, out_slice(n_chunks - 2), so0).wait()
    pltpu.make_async_copy(ob1, out_slice(n_chunks - 1), so1).wait()


@jax.jit
def kernel(input, token_values):
    n_values = input.shape[0]
    n_tokens = token_values.shape[0]
    n_workers = 32
    per_w = n_values // n_workers
    n_chunks = per_w // CHUNK

    mesh = plsc.VectorSubcoreMesh(core_axis_name="c", subcore_axis_name="s")
    k = functools.partial(
        pl.kernel,
        out_type=jax.ShapeDtypeStruct((n_values,), jnp.int32),
        mesh=mesh,
        scratch_types=[
            pltpu.VMEM((n_tokens,), jnp.float32),
            pltpu.VMEM((CHUNK,), jnp.float32),
            pltpu.VMEM((CHUNK,), jnp.float32),
            pltpu.VMEM((CHUNK,), jnp.int32),
            pltpu.VMEM((CHUNK,), jnp.int32),
            pltpu.SemaphoreType.DMA,
            pltpu.SemaphoreType.DMA,
            pltpu.SemaphoreType.DMA,
            pltpu.SemaphoreType.DMA,
        ],
        compiler_params=pltpu.CompilerParams(needs_layout_passes=False),
    )(functools.partial(_sc_body, n_tokens, per_w, n_chunks))
    return k(input, token_values)
